# Initial kernel scaffold; baseline (speedup 1.0000x reference)
import jax
import jax.numpy as jnp
from jax.experimental import pallas as pl


def kernel(x, coords, edge_index, W_se0, b_se0, g_se0, be_se0, W_se1, b_se1, W_d0, b_d0, g_d0, be_d0, W_d1, b_d1, W1, b1, W2, b2, Wfc, bfc):
    raise NotImplementedError("write your pallas kernel here")


# speedup vs baseline: 9.3599x; 9.3599x over previous
import functools

import jax
import jax.numpy as jnp
from jax import lax
from jax.experimental import pallas as pl
from jax.experimental.pallas import tpu as pltpu
from jax.experimental.pallas import tpu_sc as plsc

NC = 2
NS = 16
L = 16
IG = 4
EPG = IG * 128
NPAD = 16 * 6656
TB = 512

_SC_PARAMS = None


def _sc_mesh():
    return plsc.VectorSubcoreMesh(core_axis_name="c", subcore_axis_name="s",
                                  num_cores=NC, num_subcores=NS)


def _sc_compiler_params():
    return pltpu.CompilerParams(use_tc_tiling_on_sc=False)


def _sc_degree(dst2d):
    R = dst2d.shape[0]
    R_sc = R // NC
    RT = R_sc // NS
    GROUPS = RT // IG
    SL = NPAD // NS

    @functools.partial(
        pl.kernel,
        out_type=jax.ShapeDtypeStruct((NC, NPAD, L), jnp.float32),
        mesh=_sc_mesh(),
        scratch_types=[
            pltpu.VMEM_SHARED((NPAD, L), jnp.float32),
            pltpu.VMEM((EPG, L), jnp.float32),
            pltpu.VMEM((IG, 128), jnp.int32),
        ],
        compiler_params=_sc_compiler_params(),
    )
    def deg_kernel(dst_hbm, out_hbm, acc, ones, dstv):
        c = lax.axis_index("c")
        s = lax.axis_index("s")

        @pl.loop(0, EPG)
        def _zero(i):
            ones[i, :] = jnp.zeros((L,), jnp.float32)

        for k in range(SL // EPG):
            pltpu.sync_copy(ones, acc.at[pl.ds(s * SL + k * EPG, EPG)])

        @pl.loop(0, EPG)
        def _fill(i):
            ones[i, :] = jnp.full((L,), 1.0, jnp.float32)

        plsc.subcore_barrier()

        base = c * R_sc + s * RT

        @pl.loop(0, GROUPS)
        def _edges(g):
            r0 = base + g * IG
            pltpu.sync_copy(dst_hbm.at[pl.ds(r0, IG)], dstv)
            for j in range(IG):
                pltpu.sync_copy(ones.at[pl.ds(j * 128, 128)],
                                acc.at[dstv.at[j]], add=True)

        plsc.subcore_barrier()
        pltpu.sync_copy(acc.at[pl.ds(s * SL, SL)],
                        out_hbm.at[c].at[pl.ds(s * SL, SL)])

    return deg_kernel(dst2d)


def _sc_edge_agg(table, src2d, dst2d):
    R = src2d.shape[0]
    RT = R // NS
    GROUPS = RT // IG
    SL = NPAD // NS

    @functools.partial(
        pl.kernel,
        out_type=jax.ShapeDtypeStruct((4, NPAD, L), jnp.float32),
        mesh=_sc_mesh(),
        scratch_types=[
            pltpu.VMEM_SHARED((NPAD, L), jnp.float32),
            pltpu.VMEM((EPG, L), jnp.float32),
            pltpu.VMEM((IG, 128), jnp.int32),
            pltpu.VMEM((IG, 128), jnp.int32),
            pltpu.SemaphoreType.DMA,
        ],
        compiler_params=_sc_compiler_params(),
    )
    def agg_kernel(table_hbm, src_hbm, dst_hbm, out_hbm,
                   acc, rows, srcv, dstv, sem):
        c = lax.axis_index("c")
        s = lax.axis_index("s")

        for p in range(2):
            q = c * 2 + p

            @pl.loop(0, EPG)
            def _zfill(i):
                rows[i, :] = jnp.zeros((L,), jnp.float32)

            for k in range(SL // EPG):
                pltpu.sync_copy(rows, acc.at[pl.ds(s * SL + k * EPG, EPG)])
            plsc.subcore_barrier()

            base = s * RT

            @pl.loop(0, GROUPS)
            def _edges(g):
                r0 = base + g * IG
                pltpu.sync_copy(src_hbm.at[pl.ds(r0, IG)], srcv)
                pltpu.sync_copy(dst_hbm.at[pl.ds(r0, IG)], dstv)
                descs = []
                for j in range(IG):
                    descs.append(pltpu.async_copy(
                        table_hbm.at[q].at[srcv.at[j]],
                        rows.at[pl.ds(j * 128, 128)], sem))
                for d in descs:
                    d.wait()
                for j in range(IG):
                    pltpu.sync_copy(rows.at[pl.ds(j * 128, 128)],
                                    acc.at[dstv.at[j]], add=True)

            plsc.subcore_barrier()
            pltpu.sync_copy(acc.at[pl.ds(s * SL, SL)],
                            out_hbm.at[q].at[pl.ds(s * SL, SL)])
            plsc.subcore_barrier()

    return agg_kernel(table, src2d, dst2d)


def _layernorm(h, g, b):
    m = jnp.mean(h, axis=-1, keepdims=True)
    v = jnp.mean((h - m) * (h - m), axis=-1, keepdims=True)
    return (h - m) * lax.rsqrt(v + 1e-5) * g + b


def _dot(a, b):
    return jnp.dot(a, b, preferred_element_type=jnp.float32)


def _tc_stage1(coords_p, x_p, dacc, W_se0, b_se0, g_se0, be_se0, W_se1, b_se1,
               W_d0, b_d0, g_d0, be_d0, W_d1, b_d1, W1x, W1e):
    grid = NPAD // TB

    def body(coords_ref, x_ref, dacc_ref, Wse0_ref, bse0_ref, gse0_ref,
             bese0_ref, Wse1_ref, bse1_ref, Wd0_ref, bd0_ref, gd0_ref,
             bed0_ref, Wd1_ref, bd1_ref, W1x_ref, W1e_ref,
             hs1c_ref, hw1d2_ref, dinv_ref):
        h = jax.nn.relu(_dot(coords_ref[...], Wse0_ref[...]) + bse0_ref[...])
        h = _layernorm(h, gse0_ref[...], bese0_ref[...])
        h = jax.nn.relu(_dot(h, Wse1_ref[...]) + bse1_ref[...])
        d = jax.nn.relu(_dot(h, Wd0_ref[...]) + bd0_ref[...])
        d = _layernorm(d, gd0_ref[...], bed0_ref[...])
        emb = jax.nn.relu(_dot(d, Wd1_ref[...]) + bd1_ref[...])
        hw1 = _dot(x_ref[...], W1x_ref[...]) + _dot(emb, W1e_ref[...])
        deg = dacc_ref[0, :, :1] + dacc_ref[1, :, :1] + 1.0
        dinv = lax.rsqrt(deg)
        hs1 = hw1 * dinv
        for q in range(4):
            hs1c_ref[q] = hs1[:, q * L:(q + 1) * L]
        hw1d2_ref[...] = hw1 * (dinv * dinv)
        dinv_ref[...] = dinv

    fullspec = lambda shape: pl.BlockSpec(shape, lambda i: (0,) * len(shape))
    return pl.pallas_call(
        body,
        grid=(grid,),
        in_specs=[
            pl.BlockSpec((TB, 2), lambda i: (i, 0)),
            pl.BlockSpec((TB, 6), lambda i: (i, 0)),
            pl.BlockSpec((NC, TB, L), lambda i: (0, i, 0)),
            fullspec((2, 128)), fullspec((1, 128)), fullspec((1, 128)),
            fullspec((1, 128)), fullspec((128, 128)), fullspec((1, 128)),
            fullspec((128, 64)), fullspec((1, 64)), fullspec((1, 64)),
            fullspec((1, 64)), fullspec((64, 16)), fullspec((1, 16)),
            fullspec((6, 64)), fullspec((16, 64)),
        ],
        out_specs=[
            pl.BlockSpec((4, TB, L), lambda i: (0, i, 0)),
            pl.BlockSpec((TB, 64), lambda i: (i, 0)),
            pl.BlockSpec((TB, 1), lambda i: (i, 0)),
        ],
        out_shape=[
            jax.ShapeDtypeStruct((4, NPAD, L), jnp.float32),
            jax.ShapeDtypeStruct((NPAD, 64), jnp.float32),
            jax.ShapeDtypeStruct((NPAD, 1), jnp.float32),
        ],
    )(coords_p, x_p, dacc, W_se0, b_se0, g_se0, be_se0, W_se1, b_se1,
      W_d0, b_d0, g_d0, be_d0, W_d1, b_d1, W1x, W1e)


def _tc_stage2(acc1, hw1d2, dinv, b1, W2):
    grid = NPAD // TB

    def body(acc_ref, hwd2_ref, dinv_ref, b1_ref, W2_ref,
             hs2c_ref, hw2d2_ref):
        accs = jnp.concatenate([acc_ref[q] for q in range(4)], axis=-1)
        dinv = dinv_ref[...]
        z1 = jax.nn.relu(accs * dinv + hwd2_ref[...] + b1_ref[...])
        hw2 = _dot(z1, W2_ref[...])
        hs2 = hw2 * dinv
        for q in range(4):
            hs2c_ref[q] = hs2[:, q * L:(q + 1) * L]
        hw2d2_ref[...] = hw2 * (dinv * dinv)

    fullspec = lambda shape: pl.BlockSpec(shape, lambda i: (0,) * len(shape))
    return pl.pallas_call(
        body,
        grid=(grid,),
        in_specs=[
            pl.BlockSpec((4, TB, L), lambda i: (0, i, 0)),
            pl.BlockSpec((TB, 64), lambda i: (i, 0)),
            pl.BlockSpec((TB, 1), lambda i: (i, 0)),
            fullspec((1, 64)), fullspec((64, 64)),
        ],
        out_specs=[
            pl.BlockSpec((4, TB, L), lambda i: (0, i, 0)),
            pl.BlockSpec((TB, 64), lambda i: (i, 0)),
        ],
        out_shape=[
            jax.ShapeDtypeStruct((4, NPAD, L), jnp.float32),
            jax.ShapeDtypeStruct((NPAD, 64), jnp.float32),
        ],
    )(acc1, hw1d2, dinv, b1, W2)


def _tc_stage3(acc2, hw2d2, dinv, b2, Wfc, bfc):
    grid = NPAD // TB

    def body(acc_ref, hwd2_ref, dinv_ref, b2_ref, Wfc_ref, bfc_ref, out_ref):
        accs = jnp.concatenate([acc_ref[q] for q in range(4)], axis=-1)
        dinv = dinv_ref[...]
        z2 = jax.nn.relu(accs * dinv + hwd2_ref[...] + b2_ref[...])
        out_ref[...] = _dot(z2, Wfc_ref[...]) + bfc_ref[...]

    fullspec = lambda shape: pl.BlockSpec(shape, lambda i: (0,) * len(shape))
    return pl.pallas_call(
        body,
        grid=(grid,),
        in_specs=[
            pl.BlockSpec((4, TB, L), lambda i: (0, i, 0)),
            pl.BlockSpec((TB, 64), lambda i: (i, 0)),
            pl.BlockSpec((TB, 1), lambda i: (i, 0)),
            fullspec((1, 64)), fullspec((64, 1)), fullspec((1, 1)),
        ],
        out_specs=pl.BlockSpec((TB, 1), lambda i: (i, 0)),
        out_shape=jax.ShapeDtypeStruct((NPAD, 1), jnp.float32),
    )(acc2, hw2d2, dinv, b2, Wfc, bfc)


def kernel(x, coords, edge_index, W_se0, b_se0, g_se0, be_se0, W_se1, b_se1,
           W_d0, b_d0, g_d0, be_d0, W_d1, b_d1, W1, b1, W2, b2, Wfc, bfc):
    n = x.shape[0]
    e = edge_index.shape[1]
    assert n < NPAD

    coords_p = jnp.zeros((NPAD, 2), jnp.float32).at[:n].set(coords)
    x_p = jnp.zeros((NPAD, 6), jnp.float32).at[:n].set(x)

    epad = ((e + 2 * NS * EPG - 1) // (2 * NS * EPG)) * (2 * NS * EPG)
    src = jnp.full((epad,), n, jnp.int32).at[:e].set(edge_index[0])
    dst = jnp.full((epad,), n, jnp.int32).at[:e].set(edge_index[1])
    src2d = src.reshape(epad // 128, 128)
    dst2d = dst.reshape(epad // 128, 128)

    row = lambda v: v.reshape(1, -1)

    dacc = _sc_degree(dst2d)
    hs1c, hw1d2, dinv = _tc_stage1(
        coords_p, x_p, dacc, W_se0, row(b_se0), row(g_se0), row(be_se0),
        W_se1, row(b_se1), W_d0, row(b_d0), row(g_d0), row(be_d0),
        W_d1, row(b_d1), W1[:6], W1[6:])

    acc1 = _sc_edge_agg(hs1c, src2d, dst2d)
    hs2c, hw2d2 = _tc_stage2(acc1, hw1d2, dinv, row(b1), W2)

    acc2 = _sc_edge_agg(hs2c, src2d, dst2d)
    out = _tc_stage3(acc2, hw2d2, dinv, row(b2), Wfc, row(bfc))

    return out[:n]


# speedup vs baseline: 12.6900x; 1.3558x over previous
import functools

import jax
import jax.numpy as jnp
from jax import lax
from jax.experimental import pallas as pl
from jax.experimental.pallas import tpu as pltpu
from jax.experimental.pallas import tpu_sc as plsc

NC = 2
NS = 16
L = 16
IG = 4
EPG = IG * 128
NPAD = 16 * 6656
TB = 512

_SC_PARAMS = None


def _sc_mesh():
    return plsc.VectorSubcoreMesh(core_axis_name="c", subcore_axis_name="s",
                                  num_cores=NC, num_subcores=NS)


def _sc_compiler_params():
    return pltpu.CompilerParams(use_tc_tiling_on_sc=False)


def _sc_degree(dst1d):
    E = dst1d.shape[0]
    E_sc = E // NC
    ET = E_sc // NS
    GROUPS = ET // EPG
    SL = NPAD // NS

    @functools.partial(
        pl.kernel,
        out_type=jax.ShapeDtypeStruct((NC, NPAD, L), jnp.float32),
        mesh=_sc_mesh(),
        scratch_types=[
            pltpu.VMEM_SHARED((NPAD, L), jnp.float32),
            pltpu.VMEM((EPG, L), jnp.float32),
            pltpu.VMEM((EPG,), jnp.int32),
            pltpu.VMEM((EPG,), jnp.int32),
        ],
        compiler_params=_sc_compiler_params(),
    )
    def deg_kernel(dst_hbm, out_hbm, acc, ones, dstv0, dstv1):
        c = lax.axis_index("c")
        s = lax.axis_index("s")

        @pl.loop(0, EPG)
        def _zero(i):
            ones[i, :] = jnp.zeros((L,), jnp.float32)

        for k in range(SL // EPG):
            pltpu.sync_copy(ones, acc.at[pl.ds(s * SL + k * EPG, EPG)])

        @pl.loop(0, EPG)
        def _fill(i):
            ones[i, :] = jnp.full((L,), 1.0, jnp.float32)

        plsc.subcore_barrier()

        base = c * E_sc + s * ET
        dstv = (dstv0, dstv1)

        @pl.loop(0, GROUPS // 2)
        def _edges(gg):
            e0 = base + gg * (2 * EPG)
            for b in range(2):
                pltpu.sync_copy(dst_hbm.at[pl.ds(e0 + b * EPG, EPG)], dstv[b])
                pltpu.sync_copy(ones, acc.at[dstv[b]], add=True)

        plsc.subcore_barrier()
        pltpu.sync_copy(acc.at[pl.ds(s * SL, SL)],
                        out_hbm.at[c].at[pl.ds(s * SL, SL)])

    return deg_kernel(dst1d)


def _sc_edge_agg(table, src1d, dst1d):
    E = src1d.shape[0]
    ET = E // NS
    GROUPS = ET // EPG
    SL = NPAD // NS

    @functools.partial(
        pl.kernel,
        out_type=jax.ShapeDtypeStruct((4, NPAD, L), jnp.float32),
        mesh=_sc_mesh(),
        scratch_types=[
            pltpu.VMEM_SHARED((NPAD, L), jnp.float32),
            pltpu.VMEM((EPG, L), jnp.float32),
            pltpu.VMEM((EPG, L), jnp.float32),
            pltpu.VMEM((EPG,), jnp.int32),
            pltpu.VMEM((EPG,), jnp.int32),
            pltpu.VMEM((EPG,), jnp.int32),
            pltpu.VMEM((EPG,), jnp.int32),
            pltpu.SemaphoreType.DMA,
            pltpu.SemaphoreType.DMA,
        ],
        compiler_params=_sc_compiler_params(),
    )
    def agg_kernel(table_hbm, src_hbm, dst_hbm, out_hbm,
                   acc, rows0, rows1, srcv0, srcv1, dstv0, dstv1,
                   sem0, sem1):
        c = lax.axis_index("c")
        s = lax.axis_index("s")
        rows = (rows0, rows1)
        srcv = (srcv0, srcv1)
        dstv = (dstv0, dstv1)
        sem = (sem0, sem1)
        base = s * ET

        for p in range(2):
            q = c * 2 + p

            @pl.loop(0, EPG)
            def _zfill(i):
                rows0[i, :] = jnp.zeros((L,), jnp.float32)

            for k in range(SL // EPG):
                pltpu.sync_copy(rows0, acc.at[pl.ds(s * SL + k * EPG, EPG)])
            plsc.subcore_barrier()

            def fire(g, b):
                e0 = base + g * EPG
                pltpu.sync_copy(src_hbm.at[pl.ds(e0, EPG)], srcv[b])
                pltpu.sync_copy(dst_hbm.at[pl.ds(e0, EPG)], dstv[b])
                pltpu.async_copy(table_hbm.at[q].at[srcv[b]], rows[b], sem[b])

            def drain_scatter(b):
                pltpu.make_async_copy(table_hbm.at[q].at[srcv[b]],
                                      rows[b], sem[b]).wait()
                pltpu.sync_copy(rows[b], acc.at[dstv[b]], add=True)

            fire(0, 0)

            @pl.loop(0, GROUPS // 2 - 1)
            def _edges(k):
                fire(2 * k + 1, 1)
                drain_scatter(0)
                fire(2 * k + 2, 0)
                drain_scatter(1)

            fire(GROUPS - 1, 1)
            drain_scatter(0)
            drain_scatter(1)

            plsc.subcore_barrier()
            pltpu.sync_copy(acc.at[pl.ds(s * SL, SL)],
                            out_hbm.at[q].at[pl.ds(s * SL, SL)])
            plsc.subcore_barrier()

    return agg_kernel(table, src1d, dst1d)


def _layernorm(h, g, b):
    m = jnp.mean(h, axis=-1, keepdims=True)
    v = jnp.mean((h - m) * (h - m), axis=-1, keepdims=True)
    return (h - m) * lax.rsqrt(v + 1e-5) * g + b


def _dot(a, b):
    return jnp.dot(a, b, preferred_element_type=jnp.float32)


def _tc_stage1(coords_p, x_p, dacc, W_se0, b_se0, g_se0, be_se0, W_se1, b_se1,
               W_d0, b_d0, g_d0, be_d0, W_d1, b_d1, W1x, W1e):
    grid = NPAD // TB

    def body(coords_ref, x_ref, dacc_ref, Wse0_ref, bse0_ref, gse0_ref,
             bese0_ref, Wse1_ref, bse1_ref, Wd0_ref, bd0_ref, gd0_ref,
             bed0_ref, Wd1_ref, bd1_ref, W1x_ref, W1e_ref,
             hs1c_ref, hw1d2_ref, dinv_ref):
        h = jax.nn.relu(_dot(coords_ref[...], Wse0_ref[...]) + bse0_ref[...])
        h = _layernorm(h, gse0_ref[...], bese0_ref[...])
        h = jax.nn.relu(_dot(h, Wse1_ref[...]) + bse1_ref[...])
        d = jax.nn.relu(_dot(h, Wd0_ref[...]) + bd0_ref[...])
        d = _layernorm(d, gd0_ref[...], bed0_ref[...])
        emb = jax.nn.relu(_dot(d, Wd1_ref[...]) + bd1_ref[...])
        hw1 = _dot(x_ref[...], W1x_ref[...]) + _dot(emb, W1e_ref[...])
        deg = dacc_ref[0, :, :1] + dacc_ref[1, :, :1] + 1.0
        dinv = lax.rsqrt(deg)
        hs1 = hw1 * dinv
        for q in range(4):
            hs1c_ref[q] = hs1[:, q * L:(q + 1) * L]
        hw1d2_ref[...] = hw1 * (dinv * dinv)
        dinv_ref[...] = dinv

    fullspec = lambda shape: pl.BlockSpec(shape, lambda i: (0,) * len(shape))
    return pl.pallas_call(
        body,
        grid=(grid,),
        in_specs=[
            pl.BlockSpec((TB, 2), lambda i: (i, 0)),
            pl.BlockSpec((TB, 6), lambda i: (i, 0)),
            pl.BlockSpec((NC, TB, L), lambda i: (0, i, 0)),
            fullspec((2, 128)), fullspec((1, 128)), fullspec((1, 128)),
            fullspec((1, 128)), fullspec((128, 128)), fullspec((1, 128)),
            fullspec((128, 64)), fullspec((1, 64)), fullspec((1, 64)),
            fullspec((1, 64)), fullspec((64, 16)), fullspec((1, 16)),
            fullspec((6, 64)), fullspec((16, 64)),
        ],
        out_specs=[
            pl.BlockSpec((4, TB, L), lambda i: (0, i, 0)),
            pl.BlockSpec((TB, 64), lambda i: (i, 0)),
            pl.BlockSpec((TB, 1), lambda i: (i, 0)),
        ],
        out_shape=[
            jax.ShapeDtypeStruct((4, NPAD, L), jnp.float32),
            jax.ShapeDtypeStruct((NPAD, 64), jnp.float32),
            jax.ShapeDtypeStruct((NPAD, 1), jnp.float32),
        ],
    )(coords_p, x_p, dacc, W_se0, b_se0, g_se0, be_se0, W_se1, b_se1,
      W_d0, b_d0, g_d0, be_d0, W_d1, b_d1, W1x, W1e)


def _tc_stage2(acc1, hw1d2, dinv, b1, W2):
    grid = NPAD // TB

    def body(acc_ref, hwd2_ref, dinv_ref, b1_ref, W2_ref,
             hs2c_ref, hw2d2_ref):
        accs = jnp.concatenate([acc_ref[q] for q in range(4)], axis=-1)
        dinv = dinv_ref[...]
        z1 = jax.nn.relu(accs * dinv + hwd2_ref[...] + b1_ref[...])
        hw2 = _dot(z1, W2_ref[...])
        hs2 = hw2 * dinv
        for q in range(4):
            hs2c_ref[q] = hs2[:, q * L:(q + 1) * L]
        hw2d2_ref[...] = hw2 * (dinv * dinv)

    fullspec = lambda shape: pl.BlockSpec(shape, lambda i: (0,) * len(shape))
    return pl.pallas_call(
        body,
        grid=(grid,),
        in_specs=[
            pl.BlockSpec((4, TB, L), lambda i: (0, i, 0)),
            pl.BlockSpec((TB, 64), lambda i: (i, 0)),
            pl.BlockSpec((TB, 1), lambda i: (i, 0)),
            fullspec((1, 64)), fullspec((64, 64)),
        ],
        out_specs=[
            pl.BlockSpec((4, TB, L), lambda i: (0, i, 0)),
            pl.BlockSpec((TB, 64), lambda i: (i, 0)),
        ],
        out_shape=[
            jax.ShapeDtypeStruct((4, NPAD, L), jnp.float32),
            jax.ShapeDtypeStruct((NPAD, 64), jnp.float32),
        ],
    )(acc1, hw1d2, dinv, b1, W2)


def _tc_stage3(acc2, hw2d2, dinv, b2, Wfc, bfc):
    grid = NPAD // TB

    def body(acc_ref, hwd2_ref, dinv_ref, b2_ref, Wfc_ref, bfc_ref, out_ref):
        accs = jnp.concatenate([acc_ref[q] for q in range(4)], axis=-1)
        dinv = dinv_ref[...]
        z2 = jax.nn.relu(accs * dinv + hwd2_ref[...] + b2_ref[...])
        out_ref[...] = _dot(z2, Wfc_ref[...]) + bfc_ref[...]

    fullspec = lambda shape: pl.BlockSpec(shape, lambda i: (0,) * len(shape))
    return pl.pallas_call(
        body,
        grid=(grid,),
        in_specs=[
            pl.BlockSpec((4, TB, L), lambda i: (0, i, 0)),
            pl.BlockSpec((TB, 64), lambda i: (i, 0)),
            pl.BlockSpec((TB, 1), lambda i: (i, 0)),
            fullspec((1, 64)), fullspec((64, 1)), fullspec((1, 1)),
        ],
        out_specs=pl.BlockSpec((TB, 1), lambda i: (i, 0)),
        out_shape=jax.ShapeDtypeStruct((NPAD, 1), jnp.float32),
    )(acc2, hw2d2, dinv, b2, Wfc, bfc)


def kernel(x, coords, edge_index, W_se0, b_se0, g_se0, be_se0, W_se1, b_se1,
           W_d0, b_d0, g_d0, be_d0, W_d1, b_d1, W1, b1, W2, b2, Wfc, bfc):
    n = x.shape[0]
    e = edge_index.shape[1]
    assert n < NPAD

    coords_p = jnp.zeros((NPAD, 2), jnp.float32).at[:n].set(coords)
    x_p = jnp.zeros((NPAD, 6), jnp.float32).at[:n].set(x)

    epad = ((e + 2 * NS * EPG - 1) // (2 * NS * EPG)) * (2 * NS * EPG)
    src = jnp.full((epad,), n, jnp.int32).at[:e].set(edge_index[0])
    dst = jnp.full((epad,), n, jnp.int32).at[:e].set(edge_index[1])

    row = lambda v: v.reshape(1, -1)

    dacc = _sc_degree(dst)
    hs1c, hw1d2, dinv = _tc_stage1(
        coords_p, x_p, dacc, W_se0, row(b_se0), row(g_se0), row(be_se0),
        W_se1, row(b_se1), W_d0, row(b_d0), row(g_d0), row(be_d0),
        W_d1, row(b_d1), W1[:6], W1[6:])

    acc1 = _sc_edge_agg(hs1c, src, dst)
    hs2c, hw2d2 = _tc_stage2(acc1, hw1d2, dinv, row(b1), W2)

    acc2 = _sc_edge_agg(hs2c, src, dst)
    out = _tc_stage3(acc2, hw2d2, dinv, row(b2), Wfc, row(bfc))

    return out[:n]


# speedup vs baseline: 15.5003x; 1.2215x over previous
import functools

import jax
import jax.numpy as jnp
from jax import lax
from jax.experimental import pallas as pl
from jax.experimental.pallas import tpu as pltpu
from jax.experimental.pallas import tpu_sc as plsc

CONV = 64
NC = 2
NS = 16
L = 16
IG = 4
EPG = IG * 128
NPAD = 16 * 6656
TB = 512

_SC_PARAMS = None


def _sc_mesh():
    return plsc.VectorSubcoreMesh(core_axis_name="c", subcore_axis_name="s",
                                  num_cores=NC, num_subcores=NS)


def _sc_compiler_params():
    return pltpu.CompilerParams(use_tc_tiling_on_sc=False)


def _sc_degree(dst1d):
    E = dst1d.shape[0]
    E_sc = E // NC
    ET = E_sc // NS
    GROUPS = ET // EPG
    SL = NPAD // NS

    @functools.partial(
        pl.kernel,
        out_type=jax.ShapeDtypeStruct((NC, NPAD, L), jnp.float32),
        mesh=_sc_mesh(),
        scratch_types=[
            pltpu.VMEM_SHARED((NPAD, L), jnp.float32),
            pltpu.VMEM((EPG, L), jnp.float32),
            pltpu.VMEM((EPG,), jnp.int32),
            pltpu.VMEM((EPG,), jnp.int32),
        ],
        compiler_params=_sc_compiler_params(),
    )
    def deg_kernel(dst_hbm, out_hbm, acc, ones, dstv0, dstv1):
        c = lax.axis_index("c")
        s = lax.axis_index("s")

        @pl.loop(0, EPG)
        def _zero(i):
            ones[i, :] = jnp.zeros((L,), jnp.float32)

        for k in range(SL // EPG):
            pltpu.sync_copy(ones, acc.at[pl.ds(s * SL + k * EPG, EPG)])

        @pl.loop(0, EPG)
        def _fill(i):
            ones[i, :] = jnp.full((L,), 1.0, jnp.float32)

        plsc.subcore_barrier()

        base = c * E_sc + s * ET
        dstv = (dstv0, dstv1)

        @pl.loop(0, GROUPS // 2)
        def _edges(gg):
            e0 = base + gg * (2 * EPG)
            for b in range(2):
                pltpu.sync_copy(dst_hbm.at[pl.ds(e0 + b * EPG, EPG)], dstv[b])
                pltpu.sync_copy(ones, acc.at[dstv[b]], add=True)

        plsc.subcore_barrier()
        pltpu.sync_copy(acc.at[pl.ds(s * SL, SL)],
                        out_hbm.at[c].at[pl.ds(s * SL, SL)])

    return deg_kernel(dst1d)


def _sc_edge_agg(table, src1d, dst1d):
    E = src1d.shape[0]
    ET = E // NS
    GROUPS = ET // EPG
    SL = NPAD // NS
    TSPAN = NPAD * 8 - 7

    @functools.partial(
        pl.kernel,
        out_type=jax.ShapeDtypeStruct((NPAD, 128), jnp.float32),
        mesh=_sc_mesh(),
        scratch_types=[
            pltpu.VMEM_SHARED((NPAD, L), jnp.float32),
            pltpu.VMEM((EPG, L), jnp.float32),
            pltpu.VMEM((EPG, L), jnp.float32),
            pltpu.VMEM((EPG,), jnp.int32),
            pltpu.VMEM((EPG,), jnp.int32),
            pltpu.VMEM((EPG,), jnp.int32),
            pltpu.VMEM((EPG,), jnp.int32),
            pltpu.SemaphoreType.DMA,
            pltpu.SemaphoreType.DMA,
        ],
        compiler_params=_sc_compiler_params(),
    )
    def agg_kernel(table_hbm, src_hbm, dst_hbm, out_hbm,
                   acc, rows0, rows1, srcv0, srcv1, dstv0, dstv1,
                   sem0, sem1):
        c = lax.axis_index("c")
        s = lax.axis_index("s")
        rows = (rows0, rows1)
        srcv = (srcv0, srcv1)
        dstv = (dstv0, dstv1)
        sem = (sem0, sem1)
        base = s * ET

        for p in range(2):
            q = c * 2 + p

            @pl.loop(0, EPG)
            def _zfill(i):
                rows0[i, :] = jnp.zeros((L,), jnp.float32)

            for k in range(SL // EPG):
                pltpu.sync_copy(rows0, acc.at[pl.ds(s * SL + k * EPG, EPG)])
            plsc.subcore_barrier()

            tab_q = table_hbm.at[pl.ds(q, TSPAN)]

            def fire(g, b):
                e0 = base + g * EPG
                pltpu.sync_copy(src_hbm.at[pl.ds(e0, EPG)], srcv[b])
                pltpu.sync_copy(dst_hbm.at[pl.ds(e0, EPG)], dstv[b])
                pltpu.async_copy(tab_q.at[srcv[b]], rows[b], sem[b])

            def drain_scatter(b):
                pltpu.make_async_copy(tab_q.at[srcv[b]],
                                      rows[b], sem[b]).wait()
                pltpu.sync_copy(rows[b], acc.at[dstv[b]], add=True)

            fire(0, 0)

            @pl.loop(0, GROUPS // 2 - 1)
            def _edges(k):
                fire(2 * k + 1, 1)
                drain_scatter(0)
                fire(2 * k + 2, 0)
                drain_scatter(1)

            fire(GROUPS - 1, 1)
            drain_scatter(0)
            drain_scatter(1)

            plsc.subcore_barrier()
            pltpu.sync_copy(acc.at[pl.ds(s * SL, SL)],
                            out_hbm.at[pl.ds(s * SL, SL), pl.ds(q * L, L)])
            plsc.subcore_barrier()

    return agg_kernel(table, src1d, dst1d)


def _layernorm(h, g, b):
    m = jnp.mean(h, axis=-1, keepdims=True)
    v = jnp.mean((h - m) * (h - m), axis=-1, keepdims=True)
    return (h - m) * lax.rsqrt(v + 1e-5) * g + b


def _dot(a, b):
    return jnp.dot(a, b, preferred_element_type=jnp.float32)


def _tc_stage1(coords_p, x_p, dacc, W_se0, b_se0, g_se0, be_se0, W_se1, b_se1,
               W_d0, b_d0, g_d0, be_d0, W_d1, b_d1, W1x, W1e):
    grid = NPAD // TB

    def body(coords_ref, x_ref, dacc_ref, Wse0_ref, bse0_ref, gse0_ref,
             bese0_ref, Wse1_ref, bse1_ref, Wd0_ref, bd0_ref, gd0_ref,
             bed0_ref, Wd1_ref, bd1_ref, W1x_ref, W1e_ref,
             hst_ref, dinv_ref):
        h = jax.nn.relu(_dot(coords_ref[...], Wse0_ref[...]) + bse0_ref[...])
        h = _layernorm(h, gse0_ref[...], bese0_ref[...])
        h = jax.nn.relu(_dot(h, Wse1_ref[...]) + bse1_ref[...])
        d = jax.nn.relu(_dot(h, Wd0_ref[...]) + bd0_ref[...])
        d = _layernorm(d, gd0_ref[...], bed0_ref[...])
        emb = jax.nn.relu(_dot(d, Wd1_ref[...]) + bd1_ref[...])
        hw1 = _dot(x_ref[...], W1x_ref[...]) + _dot(emb, W1e_ref[...])
        deg = dacc_ref[0, :, :1] + dacc_ref[1, :, :1] + 1.0
        dinv = lax.rsqrt(deg)
        hst_ref[:, :CONV] = hw1 * dinv
        hst_ref[:, CONV:] = hw1 * (dinv * dinv)
        dinv_ref[...] = dinv

    fullspec = lambda shape: pl.BlockSpec(shape, lambda i: (0,) * len(shape))
    return pl.pallas_call(
        body,
        grid=(grid,),
        in_specs=[
            pl.BlockSpec((TB, 2), lambda i: (i, 0)),
            pl.BlockSpec((TB, 6), lambda i: (i, 0)),
            pl.BlockSpec((NC, TB, L), lambda i: (0, i, 0)),
            fullspec((2, 128)), fullspec((1, 128)), fullspec((1, 128)),
            fullspec((1, 128)), fullspec((128, 128)), fullspec((1, 128)),
            fullspec((128, 64)), fullspec((1, 64)), fullspec((1, 64)),
            fullspec((1, 64)), fullspec((64, 16)), fullspec((1, 16)),
            fullspec((6, 64)), fullspec((16, 64)),
        ],
        out_specs=[
            pl.BlockSpec((TB, 128), lambda i: (i, 0)),
            pl.BlockSpec((TB, 1), lambda i: (i, 0)),
        ],
        out_shape=[
            jax.ShapeDtypeStruct((NPAD, 128), jnp.float32),
            jax.ShapeDtypeStruct((NPAD, 1), jnp.float32),
        ],
    )(coords_p, x_p, dacc, W_se0, b_se0, g_se0, be_se0, W_se1, b_se1,
      W_d0, b_d0, g_d0, be_d0, W_d1, b_d1, W1x, W1e)


def _tc_stage2(acc1, hst1, dinv, b1, W2):
    grid = NPAD // TB

    def body(acc_ref, hst_ref, dinv_ref, b1_ref, W2_ref, hst2_ref):
        dinv = dinv_ref[...]
        z1 = jax.nn.relu(acc_ref[:, :CONV] * dinv + hst_ref[:, CONV:]
                         + b1_ref[...])
        hw2 = _dot(z1, W2_ref[...])
        hst2_ref[:, :CONV] = hw2 * dinv
        hst2_ref[:, CONV:] = hw2 * (dinv * dinv)

    fullspec = lambda shape: pl.BlockSpec(shape, lambda i: (0,) * len(shape))
    return pl.pallas_call(
        body,
        grid=(grid,),
        in_specs=[
            pl.BlockSpec((TB, 128), lambda i: (i, 0)),
            pl.BlockSpec((TB, 128), lambda i: (i, 0)),
            pl.BlockSpec((TB, 1), lambda i: (i, 0)),
            fullspec((1, 64)), fullspec((64, 64)),
        ],
        out_specs=pl.BlockSpec((TB, 128), lambda i: (i, 0)),
        out_shape=jax.ShapeDtypeStruct((NPAD, 128), jnp.float32),
    )(acc1, hst1, dinv, b1, W2)


def _tc_stage3(acc2, hst2, dinv, b2, Wfc, bfc):
    grid = NPAD // TB

    def body(acc_ref, hst_ref, dinv_ref, b2_ref, Wfc_ref, bfc_ref, out_ref):
        dinv = dinv_ref[...]
        z2 = jax.nn.relu(acc_ref[:, :CONV] * dinv + hst_ref[:, CONV:]
                         + b2_ref[...])
        out_ref[...] = _dot(z2, Wfc_ref[...]) + bfc_ref[...]

    fullspec = lambda shape: pl.BlockSpec(shape, lambda i: (0,) * len(shape))
    return pl.pallas_call(
        body,
        grid=(grid,),
        in_specs=[
            pl.BlockSpec((TB, 128), lambda i: (i, 0)),
            pl.BlockSpec((TB, 128), lambda i: (i, 0)),
            pl.BlockSpec((TB, 1), lambda i: (i, 0)),
            fullspec((1, 64)), fullspec((64, 1)), fullspec((1, 1)),
        ],
        out_specs=pl.BlockSpec((TB, 1), lambda i: (i, 0)),
        out_shape=jax.ShapeDtypeStruct((NPAD, 1), jnp.float32),
    )(acc2, hst2, dinv, b2, Wfc, bfc)


def kernel(x, coords, edge_index, W_se0, b_se0, g_se0, be_se0, W_se1, b_se1,
           W_d0, b_d0, g_d0, be_d0, W_d1, b_d1, W1, b1, W2, b2, Wfc, bfc):
    n = x.shape[0]
    e = edge_index.shape[1]
    assert n < NPAD

    coords_p = jnp.zeros((NPAD, 2), jnp.float32).at[:n].set(coords)
    x_p = jnp.zeros((NPAD, 6), jnp.float32).at[:n].set(x)

    epad = ((e + 2 * NS * EPG - 1) // (2 * NS * EPG)) * (2 * NS * EPG)
    src8 = jnp.full((epad,), 8 * n, jnp.int32).at[:e].set(8 * edge_index[0])
    dst = jnp.full((epad,), n, jnp.int32).at[:e].set(edge_index[1])

    row = lambda v: v.reshape(1, -1)

    dacc = _sc_degree(dst)
    hst1, dinv = _tc_stage1(
        coords_p, x_p, dacc, W_se0, row(b_se0), row(g_se0), row(be_se0),
        W_se1, row(b_se1), W_d0, row(b_d0), row(g_d0), row(be_d0),
        W_d1, row(b_d1), W1[:6], W1[6:])

    acc1 = _sc_edge_agg(hst1.reshape(NPAD * 8, L), src8, dst)
    hst2 = _tc_stage2(acc1, hst1, dinv, row(b1), W2)

    acc2 = _sc_edge_agg(hst2.reshape(NPAD * 8, L), src8, dst)
    out = _tc_stage3(acc2, hst2, dinv, row(b2), Wfc, row(bfc))

    return out[:n]


# speedup vs baseline: 19.9922x; 1.2898x over previous
import functools

import jax
import jax.numpy as jnp
from jax import lax
from jax.experimental import pallas as pl
from jax.experimental.pallas import tpu as pltpu
from jax.experimental.pallas import tpu_sc as plsc

CONV = 64
NC = 2
NS = 16
L = 16
EPG = 512
SBG = 4
NPAD = 16 * 6400
TB = 2048

_SC_PARAMS = None


def _sc_mesh():
    return plsc.VectorSubcoreMesh(core_axis_name="c", subcore_axis_name="s",
                                  num_cores=NC, num_subcores=NS)


def _sc_compiler_params():
    return pltpu.CompilerParams(use_tc_tiling_on_sc=False)


def _zero_acc_slice(acc, buf, s, SL):
    @pl.loop(0, EPG)
    def _zero(i):
        buf[i, :] = jnp.zeros((L,), jnp.float32)

    full, rem = SL // EPG, SL % EPG
    for k in range(full):
        pltpu.sync_copy(buf, acc.at[pl.ds(s * SL + k * EPG, EPG)])
    if rem:
        pltpu.sync_copy(buf.at[pl.ds(0, rem)],
                        acc.at[pl.ds(s * SL + full * EPG, rem)])


def _sc_degree(dst2):
    RTOT = dst2.shape[0]
    R_sc = RTOT // NC
    RT = R_sc // NS
    SL = NPAD // NS

    @functools.partial(
        pl.kernel,
        out_type=jax.ShapeDtypeStruct((NC, NPAD, L), jnp.float32),
        mesh=_sc_mesh(),
        scratch_types=[
            pltpu.VMEM_SHARED((NPAD, L), jnp.float32),
            pltpu.VMEM((EPG, L), jnp.float32),
            pltpu.VMEM((2, EPG), jnp.int32),
            pltpu.SemaphoreType.DMA,
            pltpu.SemaphoreType.DMA,
        ],
        compiler_params=_sc_compiler_params(),
    )
    def deg_kernel(dst_hbm, out_hbm, acc, ones, dstv, semA, semB):
        c = lax.axis_index("c")
        s = lax.axis_index("s")

        _zero_acc_slice(acc, ones, s, SL)

        @pl.loop(0, EPG)
        def _fill(i):
            ones[i, :] = jnp.full((L,), 1.0, jnp.float32)

        plsc.subcore_barrier()

        base = c * R_sc + s * RT
        sem = (semA, semB)

        @pl.loop(0, RT // 2)
        def _edges(m):
            r0 = base + m * 2
            pltpu.sync_copy(dst_hbm.at[pl.ds(r0, 2)], dstv)
            for b in range(2):
                pltpu.async_copy(ones, acc.at[dstv.at[b]], sem[b], add=True)
            for b in range(2):
                pltpu.make_async_copy(ones, acc.at[dstv.at[b]],
                                      sem[b]).wait()

        plsc.subcore_barrier()
        pltpu.sync_copy(acc.at[pl.ds(s * SL, SL)],
                        out_hbm.at[c].at[pl.ds(s * SL, SL)])

    return deg_kernel(dst2)


def _sc_edge_agg(table, src2, dst2):
    RTOT = src2.shape[0]
    RT = RTOT // NS
    SB = RT // SBG
    SL = NPAD // NS
    TSPAN = NPAD * 8 - 7

    @functools.partial(
        pl.kernel,
        out_type=jax.ShapeDtypeStruct((NPAD, 128), jnp.float32),
        mesh=_sc_mesh(),
        scratch_types=[
            pltpu.VMEM_SHARED((NPAD, L), jnp.float32),
            pltpu.VMEM((EPG, L), jnp.float32),
            pltpu.VMEM((EPG, L), jnp.float32),
            pltpu.VMEM((SBG, EPG), jnp.int32),
            pltpu.VMEM((SBG, EPG), jnp.int32),
            pltpu.VMEM((SBG, EPG), jnp.int32),
            pltpu.SemaphoreType.DMA,
            pltpu.SemaphoreType.DMA,
            pltpu.SemaphoreType.DMA,
            pltpu.SemaphoreType.DMA,
        ],
        compiler_params=_sc_compiler_params(),
    )
    def agg_kernel(table_hbm, src_hbm, dst_hbm, out_hbm,
                   acc, rows0, rows1, srcv, dstv0, dstv1,
                   gsem0, gsem1, ssem0, ssem1):
        c = lax.axis_index("c")
        s = lax.axis_index("s")
        rows = (rows0, rows1)
        dstv = (dstv0, dstv1)
        gsem = (gsem0, gsem1)
        ssem = (ssem0, ssem1)
        base = s * RT

        for p in range(2):
            q = c * 2 + p

            _zero_acc_slice(acc, rows0, s, SL)
            plsc.subcore_barrier()

            tab = table_hbm.at[pl.ds(q, TSPAN)]

            def fetch(m, ip):
                r0 = base + m * SBG
                pltpu.sync_copy(src_hbm.at[pl.ds(r0, SBG)], srcv)
                pltpu.sync_copy(dst_hbm.at[pl.ds(r0, SBG)], dstv[ip])

            def g_fire(b, j):
                pltpu.async_copy(tab.at[srcv.at[j]], rows[b], gsem[b])

            def g_wait(b):
                pltpu.make_async_copy(tab.at[srcv.at[0]], rows[b],
                                      gsem[b]).wait()

            def s_fire(b, ip, j):
                pltpu.async_copy(rows[b], acc.at[dstv[ip].at[j]],
                                 ssem[b], add=True)

            def s_wait(b):
                pltpu.make_async_copy(rows[b], acc.at[dstv[0].at[0]],
                                      ssem[b]).wait()

            def sb_body(m, ip):
                g_wait(1)
                s_fire(1, 1 - ip, SBG - 1)
                fetch(m, ip)
                s_wait(0)
                g_fire(0, 0)
                s_wait(1)
                g_fire(1, 1)
                g_wait(0)
                s_fire(0, ip, 0)
                s_wait(0)
                g_fire(0, 2)
                g_wait(1)
                s_fire(1, ip, 1)
                s_wait(1)
                g_fire(1, 3)
                g_wait(0)
                s_fire(0, ip, 2)

            fetch(0, 0)
            g_fire(0, 0)
            g_fire(1, 1)
            g_wait(0)
            s_fire(0, 0, 0)
            s_wait(0)
            g_fire(0, 2)
            g_wait(1)
            s_fire(1, 0, 1)
            s_wait(1)
            g_fire(1, 3)
            g_wait(0)
            s_fire(0, 0, 2)

            @pl.loop(0, (SB - 1) // 2)
            def _edges(t):
                sb_body(2 * t + 1, 1)
                sb_body(2 * t + 2, 0)

            g_wait(1)
            s_fire(1, 0, SBG - 1)
            s_wait(0)
            s_wait(1)

            plsc.subcore_barrier()
            pltpu.sync_copy(acc.at[pl.ds(s * SL, SL)],
                            out_hbm.at[pl.ds(s * SL, SL), pl.ds(q * L, L)])
            plsc.subcore_barrier()

    return agg_kernel(table, src2, dst2)


def _layernorm(h, g, b):
    m = jnp.mean(h, axis=-1, keepdims=True)
    v = jnp.mean((h - m) * (h - m), axis=-1, keepdims=True)
    return (h - m) * lax.rsqrt(v + 1e-5) * g + b


def _dot(a, b):
    return jnp.dot(a, b, preferred_element_type=jnp.float32)


def _tc_stage1(coords_p, x_p, dacc, W_se0, b_se0, g_se0, be_se0, W_se1, b_se1,
               W_d0, b_d0, g_d0, be_d0, W_d1, b_d1, W1x, W1e):
    grid = NPAD // TB

    def body(coords_ref, x_ref, dacc_ref, Wse0_ref, bse0_ref, gse0_ref,
             bese0_ref, Wse1_ref, bse1_ref, Wd0_ref, bd0_ref, gd0_ref,
             bed0_ref, Wd1_ref, bd1_ref, W1x_ref, W1e_ref,
             hst_ref, dinv_ref):
        h = jax.nn.relu(_dot(coords_ref[...], Wse0_ref[...]) + bse0_ref[...])
        h = _layernorm(h, gse0_ref[...], bese0_ref[...])
        h = jax.nn.relu(_dot(h, Wse1_ref[...]) + bse1_ref[...])
        d = jax.nn.relu(_dot(h, Wd0_ref[...]) + bd0_ref[...])
        d = _layernorm(d, gd0_ref[...], bed0_ref[...])
        emb = jax.nn.relu(_dot(d, Wd1_ref[...]) + bd1_ref[...])
        hw1 = _dot(x_ref[...], W1x_ref[...]) + _dot(emb, W1e_ref[...])
        deg = dacc_ref[0, :, :1] + dacc_ref[1, :, :1] + 1.0
        dinv = lax.rsqrt(deg)
        hst_ref[:, :CONV] = hw1 * dinv
        hst_ref[:, CONV:] = hw1 * (dinv * dinv)
        dinv_ref[...] = dinv

    fullspec = lambda shape: pl.BlockSpec(shape, lambda i: (0,) * len(shape))
    return pl.pallas_call(
        body,
        grid=(grid,),
        in_specs=[
            pl.BlockSpec((TB, 2), lambda i: (i, 0)),
            pl.BlockSpec((TB, 6), lambda i: (i, 0)),
            pl.BlockSpec((NC, TB, L), lambda i: (0, i, 0)),
            fullspec((2, 128)), fullspec((1, 128)), fullspec((1, 128)),
            fullspec((1, 128)), fullspec((128, 128)), fullspec((1, 128)),
            fullspec((128, 64)), fullspec((1, 64)), fullspec((1, 64)),
            fullspec((1, 64)), fullspec((64, 16)), fullspec((1, 16)),
            fullspec((6, 64)), fullspec((16, 64)),
        ],
        out_specs=[
            pl.BlockSpec((TB, 128), lambda i: (i, 0)),
            pl.BlockSpec((TB, 1), lambda i: (i, 0)),
        ],
        out_shape=[
            jax.ShapeDtypeStruct((NPAD, 128), jnp.float32),
            jax.ShapeDtypeStruct((NPAD, 1), jnp.float32),
        ],
    )(coords_p, x_p, dacc, W_se0, b_se0, g_se0, be_se0, W_se1, b_se1,
      W_d0, b_d0, g_d0, be_d0, W_d1, b_d1, W1x, W1e)


def _tc_stage2(acc1, hst1, dinv, b1, W2):
    grid = NPAD // TB

    def body(acc_ref, hst_ref, dinv_ref, b1_ref, W2_ref, hst2_ref):
        dinv = dinv_ref[...]
        z1 = jax.nn.relu(acc_ref[:, :CONV] * dinv + hst_ref[:, CONV:]
                         + b1_ref[...])
        hw2 = _dot(z1, W2_ref[...])
        hst2_ref[:, :CONV] = hw2 * dinv
        hst2_ref[:, CONV:] = hw2 * (dinv * dinv)

    fullspec = lambda shape: pl.BlockSpec(shape, lambda i: (0,) * len(shape))
    return pl.pallas_call(
        body,
        grid=(grid,),
        in_specs=[
            pl.BlockSpec((TB, 128), lambda i: (i, 0)),
            pl.BlockSpec((TB, 128), lambda i: (i, 0)),
            pl.BlockSpec((TB, 1), lambda i: (i, 0)),
            fullspec((1, 64)), fullspec((64, 64)),
        ],
        out_specs=pl.BlockSpec((TB, 128), lambda i: (i, 0)),
        out_shape=jax.ShapeDtypeStruct((NPAD, 128), jnp.float32),
    )(acc1, hst1, dinv, b1, W2)


def _tc_stage3(acc2, hst2, dinv, b2, Wfc, bfc):
    grid = NPAD // TB

    def body(acc_ref, hst_ref, dinv_ref, b2_ref, Wfc_ref, bfc_ref, out_ref):
        dinv = dinv_ref[...]
        z2 = jax.nn.relu(acc_ref[:, :CONV] * dinv + hst_ref[:, CONV:]
                         + b2_ref[...])
        out_ref[...] = _dot(z2, Wfc_ref[...]) + bfc_ref[...]

    fullspec = lambda shape: pl.BlockSpec(shape, lambda i: (0,) * len(shape))
    return pl.pallas_call(
        body,
        grid=(grid,),
        in_specs=[
            pl.BlockSpec((TB, 128), lambda i: (i, 0)),
            pl.BlockSpec((TB, 128), lambda i: (i, 0)),
            pl.BlockSpec((TB, 1), lambda i: (i, 0)),
            fullspec((1, 64)), fullspec((64, 1)), fullspec((1, 1)),
        ],
        out_specs=pl.BlockSpec((TB, 1), lambda i: (i, 0)),
        out_shape=jax.ShapeDtypeStruct((NPAD, 1), jnp.float32),
    )(acc2, hst2, dinv, b2, Wfc, bfc)


def kernel(x, coords, edge_index, W_se0, b_se0, g_se0, be_se0, W_se1, b_se1,
           W_d0, b_d0, g_d0, be_d0, W_d1, b_d1, W1, b1, W2, b2, Wfc, bfc):
    n = x.shape[0]
    e = edge_index.shape[1]
    assert n < NPAD

    coords_p = jnp.zeros((NPAD, 2), jnp.float32).at[:n].set(coords)
    x_p = jnp.zeros((NPAD, 6), jnp.float32).at[:n].set(x)

    epad = ((e + 2 * NS * EPG - 1) // (2 * NS * EPG)) * (2 * NS * EPG)
    src8 = (jnp.full((epad,), 8 * n, jnp.int32).at[:e].set(8 * edge_index[0])
            ).reshape(epad // EPG, EPG)
    dst = (jnp.full((epad,), n, jnp.int32).at[:e].set(edge_index[1])
           ).reshape(epad // EPG, EPG)

    row = lambda v: v.reshape(1, -1)

    dacc = _sc_degree(dst)
    hst1, dinv = _tc_stage1(
        coords_p, x_p, dacc, W_se0, row(b_se0), row(g_se0), row(be_se0),
        W_se1, row(b_se1), W_d0, row(b_d0), row(g_d0), row(be_d0),
        W_d1, row(b_d1), W1[:6], W1[6:])

    acc1 = _sc_edge_agg(hst1.reshape(NPAD * 8, L), src8, dst)
    hst2 = _tc_stage2(acc1, hst1, dinv, row(b1), W2)

    acc2 = _sc_edge_agg(hst2.reshape(NPAD * 8, L), src8, dst)
    out = _tc_stage3(acc2, hst2, dinv, row(b2), Wfc, row(bfc))

    return out[:n]
